# baseline (speedup 1.0000x reference)
import functools
import jax
import jax.numpy as jnp
from jax import lax
from jax.experimental import pallas as pl
from jax.experimental.pallas import tpu as pltpu
from jax.experimental.pallas import tpu_sc as plsc


N_NODES = 1024
E_EDGES = 16384
NW = 32
CHUNK = E_EDGES // NW


WSZ = 1024
GO = 2
TSH = N_NODES * N_NODES + 128
TDUMMY = N_NODES * N_NODES + 1


def _stage_a_body(lo_hbm, hi_hbm, isrc_hbm, is2_hbm, bs2_hbm,
                  t2_hbm, ordd_hbm, mm_hbm,
                  lo_v, hi_v, isrc_v, ordd_v,
                  bw_v, iw_v, bloc_v, tkey_v, arow_v, acc_v, sem):
    N = N_NODES
    E = E_EDGES
    NCH = CHUNK // 16
    wid = lax.axis_index("s") * 2 + lax.axis_index("c")
    base = pl.multiple_of(wid * CHUNK, 8)
    pltpu.sync_copy(lo_hbm.at[pl.ds(base, CHUNK)], lo_v)
    pltpu.sync_copy(hi_hbm.at[pl.ds(base, CHUNK)], hi_v)
    pltpu.sync_copy(isrc_hbm.at[pl.ds(base, CHUNK)], isrc_v)
    pltpu.sync_copy(ordd_hbm.at[wid], ordd_v)

    def zero_body(t, carry):
        for u in range(8):
            acc_v[t * 8 + u, :] = jnp.zeros((16,), jnp.float32)
        return carry
    lax.fori_loop(0, CHUNK // 8, zero_body, 0)

    dmax = jnp.zeros((16,), jnp.int32)
    wlo = jnp.full((16,), E + WSZ, jnp.int32)
    whi = jnp.zeros((16,), jnp.int32)
    for c in range(NCH):
        lo = lo_v[pl.ds(c * 16, 16)]
        hi = hi_v[pl.ds(c * 16, 16)]
        dmax = jnp.maximum(dmax, hi - lo)
        wlo = jnp.minimum(wlo, lo)
        whi = jnp.maximum(whi, hi)
    dmax_s = jnp.int32(0)
    wlo_s = jnp.int32(E + WSZ)
    whi_s = jnp.int32(0)
    for l in range(16):
        dmax_s = jnp.maximum(dmax_s, dmax[l])
        wlo_s = jnp.minimum(wlo_s, wlo[l])
        whi_s = jnp.maximum(whi_s, whi[l])
    npass = (dmax_s + GO - 1) // GO
    wbase = wlo_s - (wlo_s % 8)
    nwin = (whi_s - wbase + WSZ - 1) // WSZ

    def win_body(rwin, _):
        cwlo = pl.multiple_of(wbase + rwin * WSZ, 8)
        pltpu.sync_copy(bs2_hbm.at[pl.ds(cwlo, WSZ)], bw_v)
        pltpu.sync_copy(is2_hbm.at[pl.ds(cwlo, WSZ)], iw_v)

        def pass_body(p, _):
            obase = p * GO
            for oo in range(GO):
                for c in range(NCH):
                    lo = lo_v[pl.ds(c * 16, 16)]
                    hi = hi_v[pl.ds(c * 16, 16)]
                    b = lo + (obase + oo)
                    bloc = b - cwlo
                    inw = (b < hi) & (bloc >= 0) & (bloc < WSZ)
                    blc = jnp.minimum(jnp.maximum(bloc, 0), WSZ - 1)
                    bloc_v[pl.ds(oo * CHUNK + c * 16, 16)] = blc
                    m = plsc.load_gather(iw_v, [blc])
                    i_a = isrc_v[pl.ds(c * 16, 16)]
                    tkey_v[pl.ds(oo * CHUNK + c * 16, 16)] = (
                        jnp.where(inw, m * N + i_a, TDUMMY))
            pltpu.async_copy(t2_hbm.at[tkey_v], arow_v, sem).wait()

            def fma_body(t, carry):
                bv = []
                for oo in range(GO):
                    bv.append(bloc_v[pl.ds(oo * CHUNK + t * 16, 16)])
                for l in range(16):
                    row = t * 16 + l
                    acc = acc_v[row, :]
                    for oo in range(GO):
                        s = bv[oo][l]
                        acc = acc + bw_v[s, :] * arow_v[oo * CHUNK + row, :]
                    acc_v[row, :] = acc
                return carry
            lax.fori_loop(0, NCH, fma_body, 0)
            return _

        lax.fori_loop(0, npass, pass_body, 0)
        return _

    lax.fori_loop(0, nwin, win_body, 0)

    cps = [pltpu.async_copy(acc_v.at[pl.ds(g * 128, 128)],
                            mm_hbm.at[ordd_v.at[g]], sem)
           for g in range(4)]
    for cp in cps:
        cp.wait()


def _stage_a(seg_lo, seg_hi, i_src, i_src_pad2, Bs_pad2, T2, ord_d2):
    mesh = plsc.VectorSubcoreMesh(core_axis_name="c", subcore_axis_name="s")
    return pl.kernel(
        _stage_a_body,
        out_type=jax.ShapeDtypeStruct((E_EDGES, 16), jnp.float32),
        mesh=mesh,
        compiler_params=pltpu.CompilerParams(
            use_tc_tiling_on_sc=False, needs_layout_passes=False),
        scratch_types=[
            pltpu.VMEM((CHUNK,), jnp.int32),
            pltpu.VMEM((CHUNK,), jnp.int32),
            pltpu.VMEM((CHUNK,), jnp.int32),
            pltpu.VMEM((4, 128), jnp.int32),
            pltpu.VMEM((WSZ, 16), jnp.float32),
            pltpu.VMEM((WSZ,), jnp.int32),
            pltpu.VMEM((GO * CHUNK,), jnp.int32),
            pltpu.VMEM((GO * CHUNK,), jnp.int32),
            pltpu.VMEM((GO * CHUNK, 16), jnp.float32),
            pltpu.VMEM((CHUNK, 16), jnp.float32),
            pltpu.SemaphoreType.DMA,
        ],
    )(seg_lo, seg_hi, i_src, i_src_pad2, Bs_pad2, T2, ord_d2)


def _mlp_body(sp_ref, c2_ref, dg_ref, mm_ref, w6a, w6b, w6c, w6d, w7, out_ref):
    acc = sp_ref[...] @ w6a[...]
    acc = acc + c2_ref[...] @ w6b[...]
    acc = acc + dg_ref[...] @ w6c[...]
    acc = acc + mm_ref[...] @ w6d[...]
    out_ref[...] = jnp.maximum(acc, 0.0) @ w7[...]


def _edge_mlp(SP, C2, DG, MM, W6, W7):
    E = SP.shape[0]
    blk = 2048
    w6a = W6[0:16]
    w6b = W6[16:32]
    w6c = W6[32:96]
    w6d = W6[96:112]
    grid = (E // blk,)
    full = lambda r, c: pl.BlockSpec((r, c), lambda e: (0, 0))
    return pl.pallas_call(
        _mlp_body,
        grid=grid,
        in_specs=[
            pl.BlockSpec((blk, 16), lambda e: (e, 0)),
            pl.BlockSpec((blk, 16), lambda e: (e, 0)),
            pl.BlockSpec((blk, 64), lambda e: (e, 0)),
            pl.BlockSpec((blk, 16), lambda e: (e, 0)),
            full(16, 128), full(16, 128), full(64, 128), full(16, 128),
            full(128, 16),
        ],
        out_specs=pl.BlockSpec((blk, 16), lambda e: (e, 0)),
        out_shape=jax.ShapeDtypeStruct((E, 16), jnp.float32),
    )(SP, C2, DG, MM, w6a, w6b, w6c, w6d, W7)


def kernel(x, edge_index, SP, batch_node, W2, W3, W4, W5, W6, W7, Wn, conv_w, conv_b):
    N = x.shape[0]
    E = SP.shape[0]
    i = edge_index[0].astype(jnp.int32)
    j = edge_index[1].astype(jnp.int32)

    key_d = j * N + i
    ord_d = jnp.argsort(key_d, stable=True)
    kd_s = key_d[ord_d]
    win_d = jnp.concatenate([kd_s[:-1] != kd_s[1:], jnp.ones((1,), bool)])
    i_s = i[ord_d]
    rs_d = jnp.searchsorted(kd_s // N, jnp.arange(N + 1, dtype=jnp.int32)).astype(jnp.int32)

    key_s = i * N + j
    ord_s = jnp.argsort(key_s, stable=True)
    ks_s = key_s[ord_s]
    win_s = jnp.concatenate([ks_s[:-1] != ks_s[1:], jnp.ones((1,), bool)])
    j_s2 = j[ord_s]
    i_s2 = i[ord_s]

    h = x @ Wn
    A_val = SP @ W4
    B_val = SP @ W5
    C2 = (SP @ W2) * (SP @ W3)

    T2 = jnp.zeros((TSH, 16), jnp.float32)
    T2 = T2.at[jnp.where(win_d, kd_s, N * N + 8)].set(A_val[ord_d])

    DG = jnp.where((i == j)[:, None], h[i], 0.0)

    B_s = B_val[ord_d] * win_d[:, None]
    j_of_a = kd_s // N
    seg_lo = rs_d[j_of_a]
    seg_hi = rs_d[j_of_a + 1]
    Bs_pad2 = jnp.concatenate(
        [B_s, jnp.zeros((WSZ + 16, 16), jnp.float32)], axis=0)
    i_src_pad2 = jnp.concatenate([i_s, jnp.zeros((WSZ + 16,), jnp.int32)])
    ord_d2 = ord_d.astype(jnp.int32).reshape(NW, 4, 128)
    MM = _stage_a(seg_lo, seg_hi, i_s, i_src_pad2, Bs_pad2, T2, ord_d2)

    edge_attr = _edge_mlp(SP, C2, DG, MM, W6, W7)

    ea_s = edge_attr[ord_s] * win_s[:, None]
    xg = x[j_s2]
    contrib = (ea_s[:, :, None] * xg[:, None, :]).reshape(E, 16 * 64)
    S2 = jax.ops.segment_sum(contrib, i_s2, num_segments=N)

    r = S2 @ conv_w.reshape(16 * 64, 64) + conv_b
    return (r, edge_attr)


# speedup vs baseline: 1.2988x; 1.2988x over previous
import functools
import jax
import jax.numpy as jnp
from jax import lax
from jax.experimental import pallas as pl
from jax.experimental.pallas import tpu as pltpu
from jax.experimental.pallas import tpu_sc as plsc


N_NODES = 1024
E_EDGES = 16384
NW = 32
CHUNK = E_EDGES // NW


WSZ = 1024
GO = 2
TSH = N_NODES * N_NODES + 128
TDUMMY = N_NODES * N_NODES + 1


def _stage_a_body(lo_hbm, hi_hbm, isrc_hbm, is2_hbm, bs2_hbm, apad_hbm,
                  t_hbm, ordd_hbm, mm_hbm,
                  t_sh,
                  lo_v, hi_v, isrc_v, ordd_v,
                  bw_v, iw_v, bloc_v, tkey_v, wc_v, arow_v, acc_v, sem):
    N = N_NODES
    E = E_EDGES
    NCH = CHUNK // 16
    wid = lax.axis_index("s") * 2 + lax.axis_index("c")
    sid = lax.axis_index("s")
    base = pl.multiple_of(wid * CHUNK, 8)
    ts = TSH // 16
    toff = pl.multiple_of(sid * ts, 8)
    pltpu.sync_copy(t_hbm.at[pl.ds(toff, ts)], t_sh.at[pl.ds(toff, ts)])
    pltpu.sync_copy(lo_hbm.at[pl.ds(base, CHUNK)], lo_v)
    pltpu.sync_copy(hi_hbm.at[pl.ds(base, CHUNK)], hi_v)
    pltpu.sync_copy(isrc_hbm.at[pl.ds(base, CHUNK)], isrc_v)
    pltpu.sync_copy(ordd_hbm.at[wid], ordd_v)
    plsc.subcore_barrier()

    def zero_body(t, carry):
        for u in range(8):
            acc_v[t * 8 + u, :] = jnp.zeros((16,), jnp.float32)
        return carry
    lax.fori_loop(0, CHUNK // 8, zero_body, 0)

    dmax = jnp.zeros((16,), jnp.int32)
    wlo = jnp.full((16,), E + WSZ, jnp.int32)
    whi = jnp.zeros((16,), jnp.int32)
    for c in range(NCH):
        lo = lo_v[pl.ds(c * 16, 16)]
        hi = hi_v[pl.ds(c * 16, 16)]
        dmax = jnp.maximum(dmax, hi - lo)
        wlo = jnp.minimum(wlo, lo)
        whi = jnp.maximum(whi, hi)
    dmax_s = jnp.int32(0)
    wlo_s = jnp.int32(E + WSZ)
    whi_s = jnp.int32(0)
    for l in range(16):
        dmax_s = jnp.maximum(dmax_s, dmax[l])
        wlo_s = jnp.minimum(wlo_s, wlo[l])
        whi_s = jnp.maximum(whi_s, whi[l])
    npass = (dmax_s + GO - 1) // GO
    wbase = wlo_s - (wlo_s % 8)
    nwin = (whi_s - wbase + WSZ - 1) // WSZ

    def win_body(rwin, _):
        cwlo = pl.multiple_of(wbase + rwin * WSZ, 8)
        pltpu.sync_copy(bs2_hbm.at[pl.ds(cwlo, WSZ)], bw_v)
        pltpu.sync_copy(is2_hbm.at[pl.ds(cwlo, WSZ)], iw_v)

        def pass_body(p, _):
            obase = p * GO
            for oo in range(GO):
                for c in range(NCH):
                    lo = lo_v[pl.ds(c * 16, 16)]
                    hi = hi_v[pl.ds(c * 16, 16)]
                    b = lo + (obase + oo)
                    bloc = b - cwlo
                    inw = (b < hi) & (bloc >= 0) & (bloc < WSZ)
                    blc = jnp.minimum(jnp.maximum(bloc, 0), WSZ - 1)
                    bloc_v[pl.ds(oo * CHUNK + c * 16, 16)] = blc
                    m = plsc.load_gather(iw_v, [blc])
                    i_a = isrc_v[pl.ds(c * 16, 16)]
                    tkey_v[pl.ds(oo * CHUNK + c * 16, 16)] = (
                        jnp.where(inw, m * N + i_a, TDUMMY))
            pltpu.async_copy(t_sh.at[tkey_v], wc_v, sem).wait()
            for k in range(GO * NCH):
                w = wc_v[pl.ds(k * 16, 16)]
                wc_v[pl.ds(k * 16, 16)] = jnp.where(w < 0, E, w)
            pltpu.async_copy(apad_hbm.at[wc_v], arow_v, sem).wait()

            def fma_body(t, carry):
                bv = []
                for oo in range(GO):
                    bv.append(bloc_v[pl.ds(oo * CHUNK + t * 16, 16)])
                for l in range(16):
                    row = t * 16 + l
                    acc = acc_v[row, :]
                    for oo in range(GO):
                        s = bv[oo][l]
                        acc = acc + bw_v[s, :] * arow_v[oo * CHUNK + row, :]
                    acc_v[row, :] = acc
                return carry
            lax.fori_loop(0, NCH, fma_body, 0)
            return _

        lax.fori_loop(0, npass, pass_body, 0)
        return _

    lax.fori_loop(0, nwin, win_body, 0)

    cps = [pltpu.async_copy(acc_v.at[pl.ds(g * 128, 128)],
                            mm_hbm.at[ordd_v.at[g]], sem)
           for g in range(4)]
    for cp in cps:
        cp.wait()


def _stage_a(seg_lo, seg_hi, i_src, i_src_pad2, Bs_pad2, A_pad, T, ord_d2):
    mesh = plsc.VectorSubcoreMesh(core_axis_name="c", subcore_axis_name="s")
    return pl.kernel(
        _stage_a_body,
        out_type=jax.ShapeDtypeStruct((E_EDGES, 16), jnp.float32),
        mesh=mesh,
        compiler_params=pltpu.CompilerParams(
            use_tc_tiling_on_sc=False, needs_layout_passes=False),
        scratch_types=[
            pltpu.VMEM_SHARED((TSH,), jnp.int32),
            pltpu.VMEM((CHUNK,), jnp.int32),
            pltpu.VMEM((CHUNK,), jnp.int32),
            pltpu.VMEM((CHUNK,), jnp.int32),
            pltpu.VMEM((4, 128), jnp.int32),
            pltpu.VMEM((WSZ, 16), jnp.float32),
            pltpu.VMEM((WSZ,), jnp.int32),
            pltpu.VMEM((GO * CHUNK,), jnp.int32),
            pltpu.VMEM((GO * CHUNK,), jnp.int32),
            pltpu.VMEM((GO * CHUNK,), jnp.int32),
            pltpu.VMEM((GO * CHUNK, 16), jnp.float32),
            pltpu.VMEM((CHUNK, 16), jnp.float32),
            pltpu.SemaphoreType.DMA,
        ],
    )(seg_lo, seg_hi, i_src, i_src_pad2, Bs_pad2, A_pad, T, ord_d2)


def _mlp_body(sp_ref, c2_ref, dg_ref, mm_ref, w6a, w6b, w6c, w6d, w7, out_ref):
    acc = sp_ref[...] @ w6a[...]
    acc = acc + c2_ref[...] @ w6b[...]
    acc = acc + dg_ref[...] @ w6c[...]
    acc = acc + mm_ref[...] @ w6d[...]
    out_ref[...] = jnp.maximum(acc, 0.0) @ w7[...]


def _edge_mlp(SP, C2, DG, MM, W6, W7):
    E = SP.shape[0]
    blk = 2048
    w6a = W6[0:16]
    w6b = W6[16:32]
    w6c = W6[32:96]
    w6d = W6[96:112]
    grid = (E // blk,)
    full = lambda r, c: pl.BlockSpec((r, c), lambda e: (0, 0))
    return pl.pallas_call(
        _mlp_body,
        grid=grid,
        in_specs=[
            pl.BlockSpec((blk, 16), lambda e: (e, 0)),
            pl.BlockSpec((blk, 16), lambda e: (e, 0)),
            pl.BlockSpec((blk, 64), lambda e: (e, 0)),
            pl.BlockSpec((blk, 16), lambda e: (e, 0)),
            full(16, 128), full(16, 128), full(64, 128), full(16, 128),
            full(128, 16),
        ],
        out_specs=pl.BlockSpec((blk, 16), lambda e: (e, 0)),
        out_shape=jax.ShapeDtypeStruct((E, 16), jnp.float32),
    )(SP, C2, DG, MM, w6a, w6b, w6c, w6d, W7)


def kernel(x, edge_index, SP, batch_node, W2, W3, W4, W5, W6, W7, Wn, conv_w, conv_b):
    N = x.shape[0]
    E = SP.shape[0]
    i = edge_index[0].astype(jnp.int32)
    j = edge_index[1].astype(jnp.int32)

    key_d = j * N + i
    ord_d = jnp.argsort(key_d, stable=True)
    kd_s = key_d[ord_d]
    win_d = jnp.concatenate([kd_s[:-1] != kd_s[1:], jnp.ones((1,), bool)])
    i_s = i[ord_d]
    rs_d = jnp.searchsorted(kd_s // N, jnp.arange(N + 1, dtype=jnp.int32)).astype(jnp.int32)

    key_s = i * N + j
    ord_s = jnp.argsort(key_s, stable=True)
    ks_s = key_s[ord_s]
    win_s = jnp.concatenate([ks_s[:-1] != ks_s[1:], jnp.ones((1,), bool)])
    j_s2 = j[ord_s]
    i_s2 = i[ord_s]

    h = x @ Wn
    A_val = SP @ W4
    B_val = SP @ W5
    C2 = (SP @ W2) * (SP @ W3)

    T = jnp.full((TSH,), -1, jnp.int32)
    T = T.at[jnp.where(win_d, kd_s, N * N + 8)].set(ord_d.astype(jnp.int32))

    DG = jnp.where((i == j)[:, None], h[i], 0.0)

    B_s = B_val[ord_d] * win_d[:, None]
    j_of_a = kd_s // N
    seg_lo = rs_d[j_of_a]
    seg_hi = rs_d[j_of_a + 1]
    Bs_pad2 = jnp.concatenate(
        [B_s, jnp.zeros((WSZ + 16, 16), jnp.float32)], axis=0)
    i_src_pad2 = jnp.concatenate([i_s, jnp.zeros((WSZ + 16,), jnp.int32)])
    A_pad = jnp.concatenate(
        [A_val, jnp.zeros((16, 16), jnp.float32)], axis=0)
    ord_d2 = ord_d.astype(jnp.int32).reshape(NW, 4, 128)
    MM = _stage_a(seg_lo, seg_hi, i_s, i_src_pad2, Bs_pad2, A_pad, T, ord_d2)

    edge_attr = _edge_mlp(SP, C2, DG, MM, W6, W7)

    ea_s = edge_attr[ord_s] * win_s[:, None]
    xg = x[j_s2]
    contrib = (ea_s[:, :, None] * xg[:, None, :]).reshape(E, 16 * 64)
    S2 = jax.ops.segment_sum(contrib, i_s2, num_segments=N)

    r = S2 @ conv_w.reshape(16 * 64, 64) + conv_b
    return (r, edge_attr)
